# baseline (speedup 1.0000x reference)
import functools

import jax
import jax.numpy as jnp
from jax import lax
from jax.experimental import pallas as pl
from jax.experimental.pallas import tpu as pltpu
from jax.experimental.pallas import tpu_sc as plsc

N_NODES = 10000
N_EDGES = 160000
H = 128

NC = 2
NS = 16
NW = NC * NS
CHUNK = 128
E_PAD = 163840
CPW = E_PAD // (NW * CHUNK)
N_PAD = 10240

_LN_EPS = 1e-5


def _layer_norm(o, g, bt):
    mu = jnp.mean(o, axis=-1, keepdims=True)
    d = o - mu
    var = jnp.mean(d * d, axis=-1, keepdims=True)
    return d * lax.rsqrt(var + _LN_EPS) * g + bt


def _dot(a, b):
    return jnp.dot(a, b, preferred_element_type=jnp.float32)


def _mlp_ln_body(x_ref, w1_ref, b1_ref, w2_ref, b2_ref, g_ref, bt_ref, o_ref):
    h = jnp.maximum(_dot(x_ref[...], w1_ref[...]) + b1_ref[...], 0.0)
    o = _dot(h, w2_ref[...]) + b2_ref[...]
    o_ref[...] = _layer_norm(o, g_ref[...], bt_ref[...])


def _encoder(x, q, block_rows):
    n, fi = x.shape
    grid = n // block_rows
    full = lambda shape: pl.BlockSpec(shape, lambda i: (0, 0))
    return pl.pallas_call(
        _mlp_ln_body,
        grid=(grid,),
        in_specs=[
            pl.BlockSpec((block_rows, fi), lambda i: (i, 0)),
            full((fi, H)),
            full((1, H)),
            full((H, H)),
            full((1, H)),
            full((1, H)),
            full((1, H)),
        ],
        out_specs=pl.BlockSpec((block_rows, H), lambda i: (i, 0)),
        out_shape=jax.ShapeDtypeStruct((n, H), jnp.float32),
    )(x, q["W1"], q["b1"].reshape(1, H), q["W2"], q["b2"].reshape(1, H),
      q["g"].reshape(1, H), q["bt"].reshape(1, H))


def _project_body(x_ref, a_ref, b_ref, p_ref, q_ref):
    x = x_ref[...]
    p_ref[...] = _dot(x, a_ref[...])
    q_ref[...] = _dot(x, b_ref[...])


def _project(x_h, w1a, w1b):
    bn = 2000
    full = lambda: pl.BlockSpec((H, H), lambda i: (0, 0))
    return pl.pallas_call(
        _project_body,
        grid=(N_NODES // bn,),
        in_specs=[pl.BlockSpec((bn, H), lambda i: (i, 0)), full(), full()],
        out_specs=[pl.BlockSpec((bn, H), lambda i: (i, 0))] * 2,
        out_shape=[jax.ShapeDtypeStruct((N_NODES, H), jnp.float32)] * 2,
    )(x_h, w1a, w1b)


def _edge_body(rp_ref, rq_ref, eh_ref, gh_ref, w1c_ref, w1d_ref, b1_ref,
               w2_ref, b2_ref, g_ref, bt_ref, enew_ref, ehn_ref, esum_ref):
    eh = eh_ref[...]
    gterm = _dot(gh_ref[...], w1d_ref[...]) + b1_ref[...]
    pre = rp_ref[...] + rq_ref[...] + _dot(eh, w1c_ref[...]) + gterm
    h = jnp.maximum(pre, 0.0)
    o = _dot(h, w2_ref[...]) + b2_ref[...]
    en = _layer_norm(o, g_ref[...], bt_ref[...])
    enew_ref[...] = en
    ehn_ref[...] = eh + en

    @pl.when(pl.program_id(0) == 0)
    def _():
        esum_ref[...] = jnp.zeros_like(esum_ref)

    esum_ref[...] += jnp.sum(en, axis=0, keepdims=True)


def _edge_block(rp, rq, e_h, g_h, qe):
    be = 4000
    grid = N_EDGES // be
    full = lambda shape: pl.BlockSpec(shape, lambda i: (0,) * len(shape))
    w1 = qe["W1"]
    return pl.pallas_call(
        _edge_body,
        grid=(grid,),
        in_specs=[
            pl.BlockSpec((be, H), lambda i: (i, 0)),
            pl.BlockSpec((be, H), lambda i: (i, 0)),
            pl.BlockSpec((be, H), lambda i: (i, 0)),
            full((1, H)),
            full((H, H)),
            full((H, H)),
            full((1, H)),
            full((H, H)),
            full((1, H)),
            full((1, H)),
            full((1, H)),
        ],
        out_specs=[
            pl.BlockSpec((be, H), lambda i: (i, 0)),
            pl.BlockSpec((be, H), lambda i: (i, 0)),
            pl.BlockSpec((1, H), lambda i: (0, 0)),
        ],
        out_shape=[
            jax.ShapeDtypeStruct((E_PAD, H), jnp.float32),
            jax.ShapeDtypeStruct((N_EDGES, H), jnp.float32),
            jax.ShapeDtypeStruct((1, H), jnp.float32),
        ],
    )(rp, rq, e_h, g_h, w1[2 * H:3 * H], w1[3 * H:], qe["b1"].reshape(1, H),
      qe["W2"], qe["b2"].reshape(1, H), qe["g"].reshape(1, H),
      qe["bt"].reshape(1, H))


def _node_body(xh_ref, parts_ref, gh_ref, w1_ref, b1_ref, w2_ref, b2_ref,
               g_ref, bt_ref, xhn_ref, xsum_ref):
    xh = xh_ref[...]
    agg = parts_ref[0] + parts_ref[1]
    gterm = _dot(gh_ref[...], w1_ref[2 * H:]) + b1_ref[...]
    pre = _dot(xh, w1_ref[:H]) + _dot(agg, w1_ref[H:2 * H]) + gterm
    h = jnp.maximum(pre, 0.0)
    o = _dot(h, w2_ref[...]) + b2_ref[...]
    xn = _layer_norm(o, g_ref[...], bt_ref[...])
    xhn_ref[...] = xh + xn

    @pl.when(pl.program_id(0) == 0)
    def _():
        xsum_ref[...] = jnp.zeros_like(xsum_ref)

    xsum_ref[...] += jnp.sum(xn, axis=0, keepdims=True)


def _node_block(x_h, parts, g_h, qn):
    bn = 2000
    grid = N_NODES // bn
    full = lambda shape: pl.BlockSpec(shape, lambda i: (0,) * len(shape))
    return pl.pallas_call(
        _node_body,
        grid=(grid,),
        in_specs=[
            pl.BlockSpec((bn, H), lambda i: (i, 0)),
            pl.BlockSpec((2, bn, H), lambda i: (0, i, 0)),
            full((1, H)),
            full((3 * H, H)),
            full((1, H)),
            full((H, H)),
            full((1, H)),
            full((1, H)),
            full((1, H)),
        ],
        out_specs=[
            pl.BlockSpec((bn, H), lambda i: (i, 0)),
            pl.BlockSpec((1, H), lambda i: (0, 0)),
        ],
        out_shape=[
            jax.ShapeDtypeStruct((N_NODES, H), jnp.float32),
            jax.ShapeDtypeStruct((1, H), jnp.float32),
        ],
    )(x_h, parts, g_h, qn["W1"], qn["b1"].reshape(1, H), qn["W2"],
      qn["b2"].reshape(1, H), qn["g"].reshape(1, H), qn["bt"].reshape(1, H))


def _global_body(xsum_ref, esum_ref, gh_ref, w1_ref, b1_ref, w2_ref, b2_ref,
                 g_ref, bt_ref, gnew_ref):
    mx = xsum_ref[...] * (1.0 / N_NODES)
    me = esum_ref[...] * (1.0 / N_EDGES)
    pre = (_dot(mx, w1_ref[:H]) + _dot(me, w1_ref[H:2 * H])
           + _dot(gh_ref[...], w1_ref[2 * H:]) + b1_ref[...])
    h = jnp.maximum(pre, 0.0)
    o = _dot(h, w2_ref[...]) + b2_ref[...]
    gnew_ref[...] = _layer_norm(o, g_ref[...], bt_ref[...])


def _global_block(xsum, esum, g_h, qg):
    full = lambda shape: pl.BlockSpec(shape, lambda: (0,) * len(shape))
    return pl.pallas_call(
        _global_body,
        in_specs=[
            full((1, H)), full((1, H)), full((1, H)), full((3 * H, H)),
            full((1, H)), full((H, H)), full((1, H)), full((1, H)),
            full((1, H)),
        ],
        out_specs=full((1, H)),
        out_shape=jax.ShapeDtypeStruct((1, H), jnp.float32),
    )(xsum, esum, g_h, qg["W1"], qg["b1"].reshape(1, H), qg["W2"],
      qg["b2"].reshape(1, H), qg["g"].reshape(1, H), qg["bt"].reshape(1, H))


def _decoder_body(xh_ref, w1_ref, b1_ref, w2_ref, b2_ref, o_ref):
    h = jnp.maximum(_dot(xh_ref[...], w1_ref[...]) + b1_ref[...], 0.0)
    o_ref[...] = _dot(h, w2_ref[...]) + b2_ref[...]


def _decoder(x_h, q):
    bn = 2000
    out_dim = q["W2"].shape[1]
    full = lambda shape: pl.BlockSpec(shape, lambda i: (0,) * len(shape))
    return pl.pallas_call(
        _decoder_body,
        grid=(N_NODES // bn,),
        in_specs=[
            pl.BlockSpec((bn, H), lambda i: (i, 0)),
            full((H, H)),
            full((1, H)),
            full((H, out_dim)),
            full((1, out_dim)),
        ],
        out_specs=pl.BlockSpec((bn, out_dim), lambda i: (i, 0)),
        out_shape=jax.ShapeDtypeStruct((N_NODES, out_dim), jnp.float32),
    )(x_h, q["W1"], q["b1"].reshape(1, H), q["W2"],
      q["b2"].reshape(1, out_dim))


def _sc_mesh():
    return plsc.VectorSubcoreMesh(
        core_axis_name="c", subcore_axis_name="s", num_cores=NC,
        num_subcores=NS)


_GBUF = 3


def _gather_kernel_body(p_hbm, q_hbm, idx_hbm, rp_hbm, rq_hbm, idxs_v, idxd_v,
                        p0, p1, p2, q0, q1, q2, sem_rp, sem_rq, sem_wp,
                        sem_wq):
    w = lax.axis_index("s") * NC + lax.axis_index("c")
    pltpu.sync_copy(idx_hbm.at[0, w], idxs_v)
    pltpu.sync_copy(idx_hbm.at[1, w], idxd_v)
    bp = [p0, p1, p2]
    bq = [q0, q1, q2]
    rdp = [None] * CPW
    rdq = [None] * CPW
    wrp = [None] * CPW
    wrq = [None] * CPW
    for j in range(_GBUF):
        rdp[j] = pltpu.async_copy(p_hbm.at[idxs_v.at[j]], bp[j], sem_rp)
        rdq[j] = pltpu.async_copy(q_hbm.at[idxd_v.at[j]], bq[j], sem_rq)
    for j in range(CPW):
        s = j % _GBUF
        nj = j + 1
        if j >= _GBUF - 1 and nj < CPW:
            k = nj % _GBUF
            wrp[j - _GBUF + 1].wait()
            rdp[nj] = pltpu.async_copy(p_hbm.at[idxs_v.at[nj]], bp[k], sem_rp)
            wrq[j - _GBUF + 1].wait()
            rdq[nj] = pltpu.async_copy(q_hbm.at[idxd_v.at[nj]], bq[k], sem_rq)
        rows = pl.ds((w * CPW + j) * CHUNK, CHUNK)
        rdp[j].wait()
        wrp[j] = pltpu.async_copy(bp[s], rp_hbm.at[rows], sem_wp)
        rdq[j].wait()
        wrq[j] = pltpu.async_copy(bq[s], rq_hbm.at[rows], sem_wq)
    for j in range(CPW - _GBUF, CPW):
        wrp[j].wait()
        wrq[j].wait()


def _sc_gather2(p, q, idx4):
    f = pl.kernel(
        _gather_kernel_body,
        out_type=[jax.ShapeDtypeStruct((E_PAD, H), jnp.float32)] * 2,
        mesh=_sc_mesh(),
        scratch_types=[
            pltpu.VMEM((CPW, CHUNK), jnp.int32),
            pltpu.VMEM((CPW, CHUNK), jnp.int32),
            pltpu.VMEM((CHUNK, H), jnp.float32),
            pltpu.VMEM((CHUNK, H), jnp.float32),
            pltpu.VMEM((CHUNK, H), jnp.float32),
            pltpu.VMEM((CHUNK, H), jnp.float32),
            pltpu.VMEM((CHUNK, H), jnp.float32),
            pltpu.VMEM((CHUNK, H), jnp.float32),
            pltpu.SemaphoreType.DMA,
            pltpu.SemaphoreType.DMA,
            pltpu.SemaphoreType.DMA,
            pltpu.SemaphoreType.DMA,
        ],
    )
    return f(p, q, idx4)


_ROWS_PER_TILE = N_PAD // NS


def _scatter_kernel_body(enew_hbm, dst_hbm, parts_hbm, idx_v, c0, c1, acc,
                         sem_r, sem_s):
    cid = lax.axis_index("c")
    sid = lax.axis_index("s")
    w = sid * NC + cid
    bufc = [c0, c1]

    pltpu.sync_copy(dst_hbm.at[w], idx_v)
    rd = [None] * CPW
    sc = [None] * CPW

    def read(j):
        rows = pl.ds((w * CPW + j) * CHUNK, CHUNK)
        return pltpu.async_copy(enew_hbm.at[rows], bufc[j % 2], sem_r)

    zero = jnp.zeros((16,), jnp.float32)

    def zrow(r, carry):
        for k in range(H // 16):
            c0[r, pl.ds(k * 16, 16)] = zero
        return carry

    lax.fori_loop(0, CHUNK, zrow, 0)
    base = sid * _ROWS_PER_TILE
    for k in range(_ROWS_PER_TILE // CHUNK):
        pltpu.sync_copy(c0, acc.at[pl.ds(base + k * CHUNK, CHUNK)])
    rd[0] = read(0)
    rd[1] = read(1)
    plsc.subcore_barrier()

    for j in range(CPW):
        rd[j].wait()
        sc[j] = pltpu.async_copy(bufc[j % 2], acc.at[idx_v.at[j]], sem_s,
                                 add=True)
        if j >= 1:
            sc[j - 1].wait()
            if j + 1 < CPW:
                rd[j + 1] = read(j + 1)
    sc[CPW - 1].wait()
    plsc.subcore_barrier()

    pltpu.sync_copy(acc.at[pl.ds(base, _ROWS_PER_TILE)],
                    parts_hbm.at[cid, pl.ds(base, _ROWS_PER_TILE)])


def _sc_scatter(e_new, dst3):
    f = pl.kernel(
        _scatter_kernel_body,
        out_type=jax.ShapeDtypeStruct((2, N_PAD, H), jnp.float32),
        mesh=_sc_mesh(),
        scratch_types=[
            pltpu.VMEM((CPW, CHUNK), jnp.int32),
            pltpu.VMEM((CHUNK, H), jnp.float32),
            pltpu.VMEM((CHUNK, H), jnp.float32),
            pltpu.VMEM_SHARED((N_PAD, H), jnp.float32),
            pltpu.SemaphoreType.DMA,
            pltpu.SemaphoreType.DMA,
        ],
    )
    return f(e_new, dst3)


def kernel(x, edge_index, edge_attr, global_attr, params):
    pad = E_PAD - N_EDGES
    zpad = jnp.zeros((pad,), jnp.int32)
    idx4 = jnp.stack([
        jnp.concatenate([edge_index[0], zpad]),
        jnp.concatenate([edge_index[1], zpad]),
    ]).reshape(2, NW, CPW, CHUNK)
    dst3 = jnp.concatenate(
        [edge_index[1], jnp.full((pad,), N_NODES, jnp.int32)]).reshape(
            NW, CPW, CHUNK)

    x_h = _encoder(x, params["enc_node"], 2000)
    e_h = _encoder(edge_attr, params["enc_edge"], 4000)
    g_h = _encoder(global_attr, params["enc_glob"], 1)

    L = params["eb"]["W1"].shape[0]
    for i in range(L):
        qe = {k: v[i] for k, v in params["eb"].items()}
        qn = {k: v[i] for k, v in params["nb"].items()}
        qg = {k: v[i] for k, v in params["gb"].items()}

        p, q = _project(x_h, qe["W1"][:H], qe["W1"][H:2 * H])
        rp, rq = _sc_gather2(p, q, idx4)
        e_new, e_h, esum = _edge_block(rp, rq, e_h, g_h, qe)
        parts = _sc_scatter(e_new, dst3)
        x_h, xsum = _node_block(x_h, parts, g_h, qn)
        g_h = _global_block(xsum, esum, g_h, qg)

    return _decoder(x_h, params["dec"])


# speedup vs baseline: 1.0182x; 1.0182x over previous
import functools

import jax
import jax.numpy as jnp
from jax import lax
from jax.experimental import pallas as pl
from jax.experimental.pallas import tpu as pltpu
from jax.experimental.pallas import tpu_sc as plsc

N_NODES = 10000
N_EDGES = 160000
H = 128

NC = 2
NS = 16
NW = NC * NS
CHUNK = 128
E_PAD = 163840
CPW = E_PAD // (NW * CHUNK)
N_PAD = 10240

_LN_EPS = 1e-5


def _layer_norm(o, g, bt):
    mu = jnp.mean(o, axis=-1, keepdims=True)
    d = o - mu
    var = jnp.mean(d * d, axis=-1, keepdims=True)
    return d * lax.rsqrt(var + _LN_EPS) * g + bt


def _dot(a, b):
    return jnp.dot(a, b, preferred_element_type=jnp.float32)


def _mlp_ln_body(x_ref, w1_ref, b1_ref, w2_ref, b2_ref, g_ref, bt_ref, o_ref):
    h = jnp.maximum(_dot(x_ref[...], w1_ref[...]) + b1_ref[...], 0.0)
    o = _dot(h, w2_ref[...]) + b2_ref[...]
    o_ref[...] = _layer_norm(o, g_ref[...], bt_ref[...])


def _encoder(x, q, block_rows):
    n, fi = x.shape
    grid = n // block_rows
    full = lambda shape: pl.BlockSpec(shape, lambda i: (0, 0))
    return pl.pallas_call(
        _mlp_ln_body,
        grid=(grid,),
        in_specs=[
            pl.BlockSpec((block_rows, fi), lambda i: (i, 0)),
            full((fi, H)),
            full((1, H)),
            full((H, H)),
            full((1, H)),
            full((1, H)),
            full((1, H)),
        ],
        out_specs=pl.BlockSpec((block_rows, H), lambda i: (i, 0)),
        out_shape=jax.ShapeDtypeStruct((n, H), jnp.float32),
    )(x, q["W1"], q["b1"].reshape(1, H), q["W2"], q["b2"].reshape(1, H),
      q["g"].reshape(1, H), q["bt"].reshape(1, H))


def _project_body(x_ref, a_ref, b_ref, p_ref, q_ref):
    x = x_ref[...]
    p_ref[...] = _dot(x, a_ref[...])
    q_ref[...] = _dot(x, b_ref[...])


def _project(x_h, w1a, w1b):
    bn = 2000
    full = lambda: pl.BlockSpec((H, H), lambda i: (0, 0))
    return pl.pallas_call(
        _project_body,
        grid=(N_NODES // bn,),
        in_specs=[pl.BlockSpec((bn, H), lambda i: (i, 0)), full(), full()],
        out_specs=[pl.BlockSpec((bn, H), lambda i: (i, 0))] * 2,
        out_shape=[jax.ShapeDtypeStruct((N_NODES, H), jnp.float32)] * 2,
    )(x_h, w1a, w1b)


def _edge_body(rp_ref, rq_ref, eh_ref, gh_ref, w1c_ref, w1d_ref, b1_ref,
               w2_ref, b2_ref, g_ref, bt_ref, enew_ref, ehn_ref, esum_ref):
    eh = eh_ref[...]
    gterm = _dot(gh_ref[...], w1d_ref[...]) + b1_ref[...]
    pre = rp_ref[...] + rq_ref[...] + _dot(eh, w1c_ref[...]) + gterm
    h = jnp.maximum(pre, 0.0)
    o = _dot(h, w2_ref[...]) + b2_ref[...]
    en = _layer_norm(o, g_ref[...], bt_ref[...])
    enew_ref[...] = en
    ehn_ref[...] = eh + en

    @pl.when(pl.program_id(0) == 0)
    def _():
        esum_ref[...] = jnp.zeros_like(esum_ref)

    esum_ref[...] += jnp.sum(en, axis=0, keepdims=True)


def _edge_block(rp, rq, e_h, g_h, qe):
    be = 4000
    grid = N_EDGES // be
    full = lambda shape: pl.BlockSpec(shape, lambda i: (0,) * len(shape))
    w1 = qe["W1"]
    return pl.pallas_call(
        _edge_body,
        grid=(grid,),
        in_specs=[
            pl.BlockSpec((be, H), lambda i: (i, 0)),
            pl.BlockSpec((be, H), lambda i: (i, 0)),
            pl.BlockSpec((be, H), lambda i: (i, 0)),
            full((1, H)),
            full((H, H)),
            full((H, H)),
            full((1, H)),
            full((H, H)),
            full((1, H)),
            full((1, H)),
            full((1, H)),
        ],
        out_specs=[
            pl.BlockSpec((be, H), lambda i: (i, 0)),
            pl.BlockSpec((be, H), lambda i: (i, 0)),
            pl.BlockSpec((1, H), lambda i: (0, 0)),
        ],
        out_shape=[
            jax.ShapeDtypeStruct((E_PAD, H), jnp.float32),
            jax.ShapeDtypeStruct((N_EDGES, H), jnp.float32),
            jax.ShapeDtypeStruct((1, H), jnp.float32),
        ],
    )(rp, rq, e_h, g_h, w1[2 * H:3 * H], w1[3 * H:], qe["b1"].reshape(1, H),
      qe["W2"], qe["b2"].reshape(1, H), qe["g"].reshape(1, H),
      qe["bt"].reshape(1, H))


def _node_body(xh_ref, parts_ref, gh_ref, w1_ref, b1_ref, w2_ref, b2_ref,
               g_ref, bt_ref, xhn_ref, xsum_ref):
    xh = xh_ref[...]
    agg = parts_ref[0] + parts_ref[1]
    gterm = _dot(gh_ref[...], w1_ref[2 * H:]) + b1_ref[...]
    pre = _dot(xh, w1_ref[:H]) + _dot(agg, w1_ref[H:2 * H]) + gterm
    h = jnp.maximum(pre, 0.0)
    o = _dot(h, w2_ref[...]) + b2_ref[...]
    xn = _layer_norm(o, g_ref[...], bt_ref[...])
    xhn_ref[...] = xh + xn

    @pl.when(pl.program_id(0) == 0)
    def _():
        xsum_ref[...] = jnp.zeros_like(xsum_ref)

    xsum_ref[...] += jnp.sum(xn, axis=0, keepdims=True)


def _node_block(x_h, parts, g_h, qn):
    bn = 2000
    grid = N_NODES // bn
    full = lambda shape: pl.BlockSpec(shape, lambda i: (0,) * len(shape))
    return pl.pallas_call(
        _node_body,
        grid=(grid,),
        in_specs=[
            pl.BlockSpec((bn, H), lambda i: (i, 0)),
            pl.BlockSpec((2, bn, H), lambda i: (0, i, 0)),
            full((1, H)),
            full((3 * H, H)),
            full((1, H)),
            full((H, H)),
            full((1, H)),
            full((1, H)),
            full((1, H)),
        ],
        out_specs=[
            pl.BlockSpec((bn, H), lambda i: (i, 0)),
            pl.BlockSpec((1, H), lambda i: (0, 0)),
        ],
        out_shape=[
            jax.ShapeDtypeStruct((N_NODES, H), jnp.float32),
            jax.ShapeDtypeStruct((1, H), jnp.float32),
        ],
    )(x_h, parts, g_h, qn["W1"], qn["b1"].reshape(1, H), qn["W2"],
      qn["b2"].reshape(1, H), qn["g"].reshape(1, H), qn["bt"].reshape(1, H))


def _global_body(xsum_ref, esum_ref, gh_ref, w1_ref, b1_ref, w2_ref, b2_ref,
                 g_ref, bt_ref, gnew_ref):
    mx = xsum_ref[...] * (1.0 / N_NODES)
    me = esum_ref[...] * (1.0 / N_EDGES)
    pre = (_dot(mx, w1_ref[:H]) + _dot(me, w1_ref[H:2 * H])
           + _dot(gh_ref[...], w1_ref[2 * H:]) + b1_ref[...])
    h = jnp.maximum(pre, 0.0)
    o = _dot(h, w2_ref[...]) + b2_ref[...]
    gnew_ref[...] = _layer_norm(o, g_ref[...], bt_ref[...])


def _global_block(xsum, esum, g_h, qg):
    full = lambda shape: pl.BlockSpec(shape, lambda: (0,) * len(shape))
    return pl.pallas_call(
        _global_body,
        in_specs=[
            full((1, H)), full((1, H)), full((1, H)), full((3 * H, H)),
            full((1, H)), full((H, H)), full((1, H)), full((1, H)),
            full((1, H)),
        ],
        out_specs=full((1, H)),
        out_shape=jax.ShapeDtypeStruct((1, H), jnp.float32),
    )(xsum, esum, g_h, qg["W1"], qg["b1"].reshape(1, H), qg["W2"],
      qg["b2"].reshape(1, H), qg["g"].reshape(1, H), qg["bt"].reshape(1, H))


def _decoder_body(xh_ref, w1_ref, b1_ref, w2_ref, b2_ref, o_ref):
    h = jnp.maximum(_dot(xh_ref[...], w1_ref[...]) + b1_ref[...], 0.0)
    o_ref[...] = _dot(h, w2_ref[...]) + b2_ref[...]


def _decoder(x_h, q):
    bn = 2000
    out_dim = q["W2"].shape[1]
    full = lambda shape: pl.BlockSpec(shape, lambda i: (0,) * len(shape))
    return pl.pallas_call(
        _decoder_body,
        grid=(N_NODES // bn,),
        in_specs=[
            pl.BlockSpec((bn, H), lambda i: (i, 0)),
            full((H, H)),
            full((1, H)),
            full((H, out_dim)),
            full((1, out_dim)),
        ],
        out_specs=pl.BlockSpec((bn, out_dim), lambda i: (i, 0)),
        out_shape=jax.ShapeDtypeStruct((N_NODES, out_dim), jnp.float32),
    )(x_h, q["W1"], q["b1"].reshape(1, H), q["W2"],
      q["b2"].reshape(1, out_dim))


def _sc_mesh():
    return plsc.VectorSubcoreMesh(
        core_axis_name="c", subcore_axis_name="s", num_cores=NC,
        num_subcores=NS)


_GBUF = 4
_AHEAD = 2
CHUNK_G = 64
CPW_G = E_PAD // (NW * CHUNK_G)


def _gather_kernel_body(p_hbm, q_hbm, idx_hbm, rp_hbm, rq_hbm, idxs_v, idxd_v,
                        p0, p1, p2, p3, q0, q1, q2, q3, sem_rp, sem_rq,
                        sem_wp, sem_wq):
    w = lax.axis_index("s") * NC + lax.axis_index("c")
    pltpu.sync_copy(idx_hbm.at[0, w], idxs_v)
    pltpu.sync_copy(idx_hbm.at[1, w], idxd_v)
    bp = [p0, p1, p2, p3]
    bq = [q0, q1, q2, q3]
    rdp = [None] * CPW_G
    rdq = [None] * CPW_G
    wrp = [None] * CPW_G
    wrq = [None] * CPW_G
    for j in range(_GBUF):
        rdp[j] = pltpu.async_copy(p_hbm.at[idxs_v.at[j]], bp[j], sem_rp)
        rdq[j] = pltpu.async_copy(q_hbm.at[idxd_v.at[j]], bq[j], sem_rq)
    for j in range(CPW_G):
        s = j % _GBUF
        nj = j + _AHEAD
        if j >= _AHEAD and nj < CPW_G:
            k = nj % _GBUF
            wrp[j - _AHEAD].wait()
            rdp[nj] = pltpu.async_copy(p_hbm.at[idxs_v.at[nj]], bp[k], sem_rp)
            wrq[j - _AHEAD].wait()
            rdq[nj] = pltpu.async_copy(q_hbm.at[idxd_v.at[nj]], bq[k], sem_rq)
        rows = pl.ds((w * CPW_G + j) * CHUNK_G, CHUNK_G)
        rdp[j].wait()
        wrp[j] = pltpu.async_copy(bp[s], rp_hbm.at[rows], sem_wp)
        rdq[j].wait()
        wrq[j] = pltpu.async_copy(bq[s], rq_hbm.at[rows], sem_wq)
    for j in range(CPW_G - _GBUF, CPW_G):
        wrp[j].wait()
        wrq[j].wait()


def _sc_gather2(p, q, idx4):
    f = pl.kernel(
        _gather_kernel_body,
        out_type=[jax.ShapeDtypeStruct((E_PAD, H), jnp.float32)] * 2,
        mesh=_sc_mesh(),
        scratch_types=[
            pltpu.VMEM((CPW_G, CHUNK_G), jnp.int32),
            pltpu.VMEM((CPW_G, CHUNK_G), jnp.int32),
            pltpu.VMEM((CHUNK_G, H), jnp.float32),
            pltpu.VMEM((CHUNK_G, H), jnp.float32),
            pltpu.VMEM((CHUNK_G, H), jnp.float32),
            pltpu.VMEM((CHUNK_G, H), jnp.float32),
            pltpu.VMEM((CHUNK_G, H), jnp.float32),
            pltpu.VMEM((CHUNK_G, H), jnp.float32),
            pltpu.VMEM((CHUNK_G, H), jnp.float32),
            pltpu.VMEM((CHUNK_G, H), jnp.float32),
            pltpu.SemaphoreType.DMA,
            pltpu.SemaphoreType.DMA,
            pltpu.SemaphoreType.DMA,
            pltpu.SemaphoreType.DMA,
        ],
    )
    return f(p, q, idx4)


_ROWS_PER_TILE = N_PAD // NS


def _scatter_kernel_body(enew_hbm, dst_hbm, parts_hbm, idx_v, c0, c1, acc,
                         sem_r, sem_s):
    cid = lax.axis_index("c")
    sid = lax.axis_index("s")
    w = sid * NC + cid
    bufc = [c0, c1]

    pltpu.sync_copy(dst_hbm.at[w], idx_v)
    rd = [None] * CPW
    sc = [None] * CPW

    def read(j):
        rows = pl.ds((w * CPW + j) * CHUNK, CHUNK)
        return pltpu.async_copy(enew_hbm.at[rows], bufc[j % 2], sem_r)

    zero = jnp.zeros((16,), jnp.float32)

    def zrow(r, carry):
        for k in range(H // 16):
            c0[r, pl.ds(k * 16, 16)] = zero
        return carry

    lax.fori_loop(0, CHUNK, zrow, 0)
    base = sid * _ROWS_PER_TILE
    for k in range(_ROWS_PER_TILE // CHUNK):
        pltpu.sync_copy(c0, acc.at[pl.ds(base + k * CHUNK, CHUNK)])
    rd[0] = read(0)
    rd[1] = read(1)
    plsc.subcore_barrier()

    for j in range(CPW):
        rd[j].wait()
        sc[j] = pltpu.async_copy(bufc[j % 2], acc.at[idx_v.at[j]], sem_s,
                                 add=True)
        if j >= 1:
            sc[j - 1].wait()
            if j + 1 < CPW:
                rd[j + 1] = read(j + 1)
    sc[CPW - 1].wait()
    plsc.subcore_barrier()

    pltpu.sync_copy(acc.at[pl.ds(base, _ROWS_PER_TILE)],
                    parts_hbm.at[cid, pl.ds(base, _ROWS_PER_TILE)])


def _sc_scatter(e_new, dst3):
    f = pl.kernel(
        _scatter_kernel_body,
        out_type=jax.ShapeDtypeStruct((2, N_PAD, H), jnp.float32),
        mesh=_sc_mesh(),
        scratch_types=[
            pltpu.VMEM((CPW, CHUNK), jnp.int32),
            pltpu.VMEM((CHUNK, H), jnp.float32),
            pltpu.VMEM((CHUNK, H), jnp.float32),
            pltpu.VMEM_SHARED((N_PAD, H), jnp.float32),
            pltpu.SemaphoreType.DMA,
            pltpu.SemaphoreType.DMA,
        ],
    )
    return f(e_new, dst3)


def kernel(x, edge_index, edge_attr, global_attr, params):
    pad = E_PAD - N_EDGES
    zpad = jnp.zeros((pad,), jnp.int32)
    idx4 = jnp.stack([
        jnp.concatenate([edge_index[0], zpad]),
        jnp.concatenate([edge_index[1], zpad]),
    ]).reshape(2, NW, CPW_G, CHUNK_G)
    dst3 = jnp.concatenate(
        [edge_index[1], jnp.full((pad,), N_NODES, jnp.int32)]).reshape(
            NW, CPW, CHUNK)

    x_h = _encoder(x, params["enc_node"], 2000)
    e_h = _encoder(edge_attr, params["enc_edge"], 4000)
    g_h = _encoder(global_attr, params["enc_glob"], 1)

    L = params["eb"]["W1"].shape[0]
    for i in range(L):
        qe = {k: v[i] for k, v in params["eb"].items()}
        qn = {k: v[i] for k, v in params["nb"].items()}
        qg = {k: v[i] for k, v in params["gb"].items()}

        p, q = _project(x_h, qe["W1"][:H], qe["W1"][H:2 * H])
        rp, rq = _sc_gather2(p, q, idx4)
        e_new, e_h, esum = _edge_block(rp, rq, e_h, g_h, qe)
        parts = _sc_scatter(e_new, dst3)
        x_h, xsum = _node_block(x_h, parts, g_h, qn)
        g_h = _global_block(xsum, esum, g_h, qg)

    return _decoder(x_h, params["dec"])
